# Initial kernel scaffold; baseline (speedup 1.0000x reference)
import jax
import jax.numpy as jnp
from jax.experimental import pallas as pl


def kernel(x, edge_index, W1, b1, g1, be1, W2, b2, g2, be2):
    raise NotImplementedError("write your pallas kernel here")


# speedup vs baseline: 12.1277x; 12.1277x over previous
import dataclasses
import functools

import jax
import jax.numpy as jnp
from jax import lax
from jax.experimental import pallas as pl
from jax.experimental.pallas import tpu as pltpu
from jax.experimental.pallas import tpu_sc as plsc

N = 10000
D = 128
E = 320000
EPS = 1e-5

NC = 2
NS = 16
NW = NC * NS
CHUNK = 128
CH = -(-E // (NW * CHUNK))
EP = NW * CH * CHUNK
NN = 10240
RPT = NN // NS
NB = 5
RB = NN // NB

_mesh = plsc.VectorSubcoreMesh(core_axis_name="c", subcore_axis_name="s")

_cp_no_layout = pltpu.CompilerParams()
if "needs_layout_passes" in pltpu.CompilerParams.__dataclass_fields__:
    _cp_no_layout = dataclasses.replace(_cp_no_layout, needs_layout_passes=False)


@functools.partial(
    pl.kernel,
    mesh=_mesh,
    out_type=jax.ShapeDtypeStruct((NW, NN), jnp.float32),
    compiler_params=_cp_no_layout,
    scratch_types=[
        pltpu.VMEM((CH, CHUNK), jnp.int32),
        pltpu.VMEM((NN,), jnp.float32),
    ],
)
def _sc_degree(dst_hbm, out_hbm, idx_v, hist_v):
    wid = lax.axis_index("s") * NC + lax.axis_index("c")
    pltpu.sync_copy(dst_hbm.at[wid], idx_v)

    @pl.loop(0, NN, step=16)
    def _(i):
        hist_v[pl.ds(i, 16)] = jnp.zeros((16,), jnp.float32)

    ones = jnp.full((16,), 1.0, jnp.float32)

    @pl.loop(0, CH)
    def _(j):
        row = idx_v.at[j]
        for k in range(CHUNK // 16):
            plsc.addupdate_scatter(hist_v, [row[pl.ds(k * 16, 16)]], ones)

    pltpu.sync_copy(hist_v, out_hbm.at[wid])


@functools.partial(
    pl.kernel,
    mesh=_mesh,
    out_type=jax.ShapeDtypeStruct((NC, NN, D), jnp.float32),
    scratch_types=[
        pltpu.VMEM((CH, CHUNK), jnp.int32),
        pltpu.VMEM((CH, CHUNK), jnp.int32),
        pltpu.VMEM((CHUNK, D), jnp.float32),
        pltpu.VMEM_SHARED((NN, D), jnp.float32),
    ],
)
def _sc_edge(xs_hbm, src_hbm, dst_hbm, zeros_hbm, out_hbm,
             src_v, dst_v, rows_v, acc_sh):
    c = lax.axis_index("c")
    s = lax.axis_index("s")
    wid = s * NC + c

    pltpu.sync_copy(zeros_hbm, acc_sh.at[pl.ds(s * RPT, RPT)])
    pltpu.sync_copy(src_hbm.at[wid], src_v)
    pltpu.sync_copy(dst_hbm.at[wid], dst_v)
    plsc.subcore_barrier()

    @pl.loop(0, CH)
    def _(j):
        pltpu.sync_copy(xs_hbm.at[src_v.at[j]], rows_v)
        pltpu.sync_copy(rows_v, acc_sh.at[dst_v.at[j]], add=True)

    plsc.subcore_barrier()
    pltpu.sync_copy(acc_sh.at[pl.ds(s * RPT, RPT)],
                    out_hbm.at[c, pl.ds(s * RPT, RPT)])


def _tc_pre_body(x_ref, w_ref, hist_ref, xs_ref, dinv_ref):
    deg = jnp.sum(hist_ref[...], axis=0, keepdims=True)
    dinv = lax.rsqrt(1.0 + deg).reshape(RB, 1)
    hw = jnp.dot(x_ref[...], w_ref[...],
                 preferred_element_type=jnp.float32,
                 precision=lax.Precision.HIGHEST)
    xs_ref[...] = hw * dinv
    dinv_ref[...] = dinv


def _tc_pre(x, w1, hist):
    return pl.pallas_call(
        _tc_pre_body,
        grid=(NB,),
        in_specs=[
            pl.BlockSpec((RB, D), lambda i: (i, 0)),
            pl.BlockSpec((D, D), lambda i: (0, 0)),
            pl.BlockSpec((NW, RB), lambda i: (0, i)),
        ],
        out_specs=[
            pl.BlockSpec((RB, D), lambda i: (i, 0)),
            pl.BlockSpec((RB, 1), lambda i: (i, 0)),
        ],
        out_shape=[
            jax.ShapeDtypeStruct((NN, D), jnp.float32),
            jax.ShapeDtypeStruct((NN, 1), jnp.float32),
        ],
    )(x, w1, hist)


def _post_math(p, xs, dinv, b, g, be, res):
    acc = p[0] + p[1] + xs
    agg = acc * dinv + b
    mu = jnp.mean(agg, axis=1, keepdims=True)
    dev = agg - mu
    var = jnp.mean(dev * dev, axis=1, keepdims=True)
    hn = dev * lax.rsqrt(var + EPS) * g + be
    e = jnp.where(hn > 0, hn, jnp.exp(jnp.minimum(hn, 0.0)) - 1.0)
    return e + res


def _tc_mid_body(p_ref, xs_ref, x_ref, dinv_ref, b_ref, g_ref, be_ref,
                 w2_ref, h_ref, xs2_ref):
    h = _post_math(p_ref[...], xs_ref[...], dinv_ref[...],
                   b_ref[...], g_ref[...], be_ref[...], x_ref[...])
    h_ref[...] = h
    xs2_ref[...] = jnp.dot(h, w2_ref[...],
                           preferred_element_type=jnp.float32,
                           precision=lax.Precision.HIGHEST) * dinv_ref[...]


def _tc_mid(p, xs, x, dinv, b1, g1, be1, w2):
    return pl.pallas_call(
        _tc_mid_body,
        grid=(NB,),
        in_specs=[
            pl.BlockSpec((2, RB, D), lambda i: (0, i, 0)),
            pl.BlockSpec((RB, D), lambda i: (i, 0)),
            pl.BlockSpec((RB, D), lambda i: (i, 0)),
            pl.BlockSpec((RB, 1), lambda i: (i, 0)),
            pl.BlockSpec((1, D), lambda i: (0, 0)),
            pl.BlockSpec((1, D), lambda i: (0, 0)),
            pl.BlockSpec((1, D), lambda i: (0, 0)),
            pl.BlockSpec((D, D), lambda i: (0, 0)),
        ],
        out_specs=[
            pl.BlockSpec((RB, D), lambda i: (i, 0)),
            pl.BlockSpec((RB, D), lambda i: (i, 0)),
        ],
        out_shape=[
            jax.ShapeDtypeStruct((NN, D), jnp.float32),
            jax.ShapeDtypeStruct((NN, D), jnp.float32),
        ],
    )(p, xs, x, dinv, b1, g1, be1, w2)


def _tc_post_body(p_ref, xs_ref, h1_ref, dinv_ref, b_ref, g_ref, be_ref,
                  out_ref):
    out_ref[...] = _post_math(p_ref[...], xs_ref[...], dinv_ref[...],
                              b_ref[...], g_ref[...], be_ref[...], h1_ref[...])


def _tc_post(p, xs, h1, dinv, b2, g2, be2):
    return pl.pallas_call(
        _tc_post_body,
        grid=(NB,),
        in_specs=[
            pl.BlockSpec((2, RB, D), lambda i: (0, i, 0)),
            pl.BlockSpec((RB, D), lambda i: (i, 0)),
            pl.BlockSpec((RB, D), lambda i: (i, 0)),
            pl.BlockSpec((RB, 1), lambda i: (i, 0)),
            pl.BlockSpec((1, D), lambda i: (0, 0)),
            pl.BlockSpec((1, D), lambda i: (0, 0)),
            pl.BlockSpec((1, D), lambda i: (0, 0)),
        ],
        out_specs=pl.BlockSpec((RB, D), lambda i: (i, 0)),
        out_shape=jax.ShapeDtypeStruct((NN, D), jnp.float32),
    )(p, xs, h1, dinv, b2, g2, be2)


def kernel(x, edge_index, W1, b1, g1, be1, W2, b2, g2, be2):
    pad = EP - E
    src = jnp.concatenate([edge_index[0], jnp.zeros((pad,), jnp.int32)])
    dst = jnp.concatenate([edge_index[1], jnp.full((pad,), N, jnp.int32)])
    srcp = src.reshape(NW, CH, CHUNK)
    dstp = dst.reshape(NW, CH, CHUNK)
    zeros = jnp.zeros((RPT, D), jnp.float32)
    xp = jnp.concatenate([x, jnp.zeros((NN - N, D), jnp.float32)])

    hist = _sc_degree(dstp)

    xs1, dinv = _tc_pre(xp, W1, hist)
    p1 = _sc_edge(xs1, srcp, dstp, zeros)
    h1, xs2 = _tc_mid(p1, xs1, xp, dinv,
                      b1.reshape(1, D), g1.reshape(1, D), be1.reshape(1, D),
                      W2)
    p2 = _sc_edge(xs2, srcp, dstp, zeros)
    h2 = _tc_post(p2, xs2, h1, dinv,
                  b2.reshape(1, D), g2.reshape(1, D), be2.reshape(1, D))
    return h2[:N]
